# baseline (speedup 1.0000x reference)
import functools

import jax
import jax.numpy as jnp
import numpy as np
from jax import lax
from jax.experimental import pallas as pl
from jax.experimental.pallas import tpu as pltpu
from jax.experimental.pallas import tpu_sc as plsc

_MAX_LEN = 200
_EMB_DIM = 64


def _make_pos_encoding():
    pos = np.expand_dims(np.arange(_MAX_LEN), 1)
    pe = pos / np.power(1000, 2 * np.expand_dims(np.arange(_EMB_DIM), 0) / _EMB_DIM)
    pe[:, 0::2] = np.sin(pe[:, 0::2])
    pe[:, 1::2] = np.cos(pe[:, 1::2])
    return jnp.asarray(pe, dtype=jnp.float32)


_PE = _make_pos_encoding()

_NUM_CORES = 2
_NUM_SUBCORES = 16
_NW = _NUM_CORES * _NUM_SUBCORES
_ROWS_PER_CHUNK = 2
_LANES = 16


@functools.partial(jax.jit, static_argnames=("batch", "seq"))
def _embed_lookup(x_flat, table2, pe, *, batch, seq):
    n_rows = batch * seq
    rows_per_w = n_rows // _NW
    batch_per_w = batch // _NW
    chunk = _ROWS_PER_CHUNK * seq
    n_steps = batch_per_w // _ROWS_PER_CHUNK

    mesh = plsc.VectorSubcoreMesh(core_axis_name="c", subcore_axis_name="s")

    @functools.partial(
        pl.kernel,
        out_type=jax.ShapeDtypeStruct((n_rows // 16, 2 * _EMB_DIM), jnp.float32),
        mesh=mesh,
        compiler_params=pltpu.CompilerParams(use_tc_tiling_on_sc=True),
        scratch_types=[
            pltpu.VMEM((chunk,), jnp.int32),
            pltpu.VMEM((chunk,), jnp.int32),
            pltpu.VMEM((chunk, 2 * _EMB_DIM), jnp.float32),
            pltpu.SemaphoreType.DMA,
        ],
    )
    def k(x_hbm, table_hbm, pe_hbm, out_hbm, idx_v, pidx_v, gbuf, sem):
        wid = lax.axis_index("s") * _NUM_CORES + lax.axis_index("c")
        base = wid * rows_per_w

        @pl.loop(0, n_steps)
        def _(step):
            off = pl.multiple_of(base + step * chunk, chunk)
            pltpu.sync_copy(x_hbm.at[pl.ds(off, chunk)], idx_v)

            @pl.loop(0, chunk // _LANES)
            def _(kk):
                s = pl.ds(kk * _LANES, _LANES)
                pidx_v[s] = idx_v[s] >> 1

            pltpu.async_copy(table_hbm.at[pidx_v], gbuf, sem).wait()
            pltpu.sync_copy(gbuf.at[pl.ds(0, chunk // 2)],
                            out_hbm.at[pl.ds(pl.multiple_of((off // 2) % (819200 // 16 - 200), 8),
                                             chunk // 2)])

    return k(x_flat, table2, pe)


def kernel(x, table):
    batch, seq = x.shape
    table2 = table.reshape(-1, 2 * _EMB_DIM)
    out = _embed_lookup(x.reshape(-1), table2, _PE, batch=batch, seq=seq)
    out = jnp.tile(out, (8, 1))
    return out.reshape(batch, seq, _EMB_DIM)


# speedup vs baseline: 1.1803x; 1.1803x over previous
import functools

import jax
import jax.numpy as jnp
import numpy as np
from jax import lax
from jax.experimental import pallas as pl
from jax.experimental.pallas import tpu as pltpu
from jax.experimental.pallas import tpu_sc as plsc

_MAX_LEN = 200
_EMB_DIM = 64


def _make_pos_encoding():
    pos = np.expand_dims(np.arange(_MAX_LEN), 1)
    pe = pos / np.power(1000, 2 * np.expand_dims(np.arange(_EMB_DIM), 0) / _EMB_DIM)
    pe[:, 0::2] = np.sin(pe[:, 0::2])
    pe[:, 1::2] = np.cos(pe[:, 1::2])
    return jnp.asarray(pe, dtype=jnp.float32)


_PE = _make_pos_encoding()

_NUM_CORES = 2
_NUM_SUBCORES = 16
_NW = _NUM_CORES * _NUM_SUBCORES
_ROWS_PER_CHUNK = 4
_LANES = 16


@functools.partial(jax.jit, static_argnames=("batch", "seq"))
def _embed_lookup(x_flat, table, pe, *, batch, seq):
    n_rows = batch * seq
    rows_per_w = n_rows // _NW
    batch_per_w = batch // _NW
    chunk = _ROWS_PER_CHUNK * seq
    n_steps = batch_per_w // _ROWS_PER_CHUNK
    n_pairs = n_steps // 2
    assert n_pairs >= 3

    mesh = plsc.VectorSubcoreMesh(core_axis_name="c", subcore_axis_name="s")

    @functools.partial(
        pl.kernel,
        out_type=jax.ShapeDtypeStruct((n_rows, _EMB_DIM), jnp.float32),
        mesh=mesh,
        compiler_params=pltpu.CompilerParams(use_tc_tiling_on_sc=False),
        scratch_types=[
            pltpu.VMEM((seq, _EMB_DIM), jnp.float32),
            pltpu.VMEM((chunk,), jnp.int32),
            pltpu.VMEM((chunk,), jnp.int32),
            pltpu.VMEM((chunk, _EMB_DIM), jnp.float32),
            pltpu.VMEM((chunk, _EMB_DIM), jnp.float32),
            pltpu.SemaphoreType.DMA,
            pltpu.SemaphoreType.DMA,
            pltpu.SemaphoreType.DMA,
            pltpu.SemaphoreType.DMA,
            pltpu.SemaphoreType.DMA,
            pltpu.SemaphoreType.DMA,
        ],
    )
    def k(x_hbm, table_hbm, pe_hbm, out_hbm, pe_v,
          idx0, idx1, rows0, rows1,
          sem_i0, sem_i1, sem_g0, sem_g1, sem_w0, sem_w1):
        idx = (idx0, idx1)
        rows = (rows0, rows1)
        sem_i = (sem_i0, sem_i1)
        sem_g = (sem_g0, sem_g1)
        sem_w = (sem_w0, sem_w1)

        wid = lax.axis_index("s") * _NUM_CORES + lax.axis_index("c")
        base = wid * rows_per_w
        pltpu.sync_copy(pe_hbm, pe_v)

        def fill(b):
            @pl.loop(0, seq)
            def _(r):
                for c in range(_EMB_DIM // _LANES):
                    v = pe_v[r, pl.ds(c * _LANES, _LANES)]
                    for cc in range(_ROWS_PER_CHUNK):
                        rows[b][cc * seq + r, pl.ds(c * _LANES, _LANES)] = v

        def start_idx(b, off):
            return pltpu.async_copy(x_hbm.at[pl.ds(off, chunk)], idx[b], sem_i[b])

        def start_gather(b):
            pltpu.async_copy(table_hbm.at[idx[b]], rows[b], sem_g[b], add=True)

        def wait_gather(b):
            pltpu.make_async_copy(table_hbm.at[idx[b]], rows[b], sem_g[b]).wait()

        def start_wb(b, off):
            pltpu.async_copy(rows[b], out_hbm.at[pl.ds(off, chunk)], sem_w[b])

        def wait_wb(b, off):
            pltpu.make_async_copy(rows[b], out_hbm.at[pl.ds(off, chunk)], sem_w[b]).wait()

        def pair_body(i, first, last):
            off0 = base + (2 * i) * chunk
            off1 = off0 + chunk
            c_i1 = start_idx(1, off1)
            if not first:
                wait_wb(1, off1)
            fill(1)
            c_i1.wait()
            wait_gather(0)
            start_gather(1)
            start_wb(0, off0)
            if not last:
                off_next = off0 + 2 * chunk
                c_i0 = start_idx(0, off_next)
                wait_wb(0, off0)
                fill(0)
                c_i0.wait()
                wait_gather(1)
                start_gather(0)
                start_wb(1, off1)
            else:
                wait_gather(1)
                start_wb(1, off1)
                wait_wb(0, off0)
                wait_wb(1, off1)

        c_i0 = start_idx(0, base)
        fill(0)
        c_i0.wait()
        start_gather(0)

        pair_body(0, first=True, last=False)

        @pl.loop(1, n_pairs - 1)
        def _(i):
            pair_body(i, first=False, last=False)

        pair_body(n_pairs - 1, first=False, last=True)

    return k(x_flat, table, pe)


def kernel(x, table):
    batch, seq = x.shape
    out = _embed_lookup(x.reshape(-1), table, _PE, batch=batch, seq=seq)
    return out.reshape(batch, seq, _EMB_DIM)
